# baseline (speedup 1.0000x reference)
import functools

import jax
import jax.numpy as jnp
from jax import lax
from jax.experimental import pallas as pl
from jax.experimental.pallas import tpu as pltpu
from jax.experimental.pallas import tpu_sc as plsc

_BB = 256
_NTILES = 32
_BPT = 16
_B_SC = _NTILES * _BPT


def _gin_block(h_ref, w_ref, v_ref, lw_ref, lb_ref, out_ref):
    hb = h_ref[:]
    bb, n, d = hb.shape
    feat = jnp.dot(
        hb.reshape(bb * n, d), w_ref[:], preferred_element_type=jnp.float32
    )
    feat = feat.reshape(bb, n, -1)
    k = n
    while k > 1:
        k //= 2
        feat = feat[:, :k, :] * feat[:, k:, :]
    pooled = feat[:, 0, :]
    m = jnp.dot(lw_ref[:], v_ref[:], preferred_element_type=jnp.float32)
    out_ref[:] = (
        jnp.dot(pooled, m.T, preferred_element_type=jnp.float32) + lb_ref[:]
    )


def _mt_block(v_ref, lw_ref, out_ref):
    out_ref[:] = lax.dot_general(
        v_ref[:], lw_ref[:], (((0,), (1,)), ((), ())),
        preferred_element_type=jnp.float32,
    )


def _sc_body(h_hbm, wb_hbm, mt_hbm, lb_hbm, out_hbm,
             hbuf, wbv, mtv, lbv, sbuf, sc16, sem, semw):
    c = lax.axis_index("c")
    s = lax.axis_index("s")
    wid = s * 2 + c
    base = wid * _BPT

    pltpu.async_copy(wb_hbm, wbv, semw).wait()
    pltpu.async_copy(mt_hbm, mtv, semw).wait()
    pltpu.async_copy(lb_hbm, lbv, semw).wait()

    lane = lax.iota(jnp.int32, 16)
    base_g = [lane * 128 + g * 2048 for g in range(4)]
    rot = [(lane + k) & 15 for k in (8, 4, 2, 1)]

    def _round_bf16(x):
        u = plsc.bitcast(x, jnp.uint32)
        u = (u + jnp.uint32(0x7FFF) + ((u >> 16) & jnp.uint32(1))) & jnp.uint32(
            0xFFFF0000
        )
        return plsc.bitcast(u, jnp.float32)

    def d_body(d, accs):
        hv = [_round_bf16(plsc.load_gather(hbuf, [bg + d])) for bg in base_g]
        new = list(accs)
        for r in range(8):
            wv = wbv[pl.ds(d * 128 + r * 16, 16)]
            for g in range(4):
                new[g * 8 + r] = new[g * 8 + r] + hv[g] * wv
        return tuple(new)

    def b_body(i, carry):
        pltpu.async_copy(h_hbm.at[base + i], hbuf, sem).wait()
        accs = lax.fori_loop(
            0, 128, d_body, tuple(jnp.zeros((16,), jnp.float32) for _ in range(32))
        )
        pvecs = []
        for r in range(8):
            q = accs[r] * accs[8 + r] * accs[16 + r] * accs[24 + r]
            for ridx in rot:
                sc16[...] = q
                q = q * plsc.load_gather(sc16, [ridx])
            pvecs.append(q)
        for og in range(8):
            sv = lbv[pl.ds(og * 16, 16)]
            for r in range(8):
                sv = sv + pvecs[r] * mtv[pl.ds(r * 128 + og * 16, 16)]
            sbuf[pl.ds(i * 128 + og * 16, 16)] = sv
        return carry

    lax.fori_loop(0, _BPT, b_body, 0)
    pltpu.async_copy(sbuf, out_hbm.at[pl.ds(base * 128, _BPT * 128)], semw).wait()


def _sc_scores(h, w_bcast, mt, lin_b):
    f = functools.partial(
        pl.kernel,
        out_type=jax.ShapeDtypeStruct((_B_SC * 128,), jnp.float32),
        mesh=plsc.VectorSubcoreMesh(core_axis_name="c", subcore_axis_name="s"),
        compiler_params=pltpu.CompilerParams(needs_layout_passes=False),
        scratch_types=[
            pltpu.VMEM((64 * 128,), jnp.float32),
            pltpu.VMEM((128 * 8 * 16,), jnp.float32),
            pltpu.VMEM((8 * 128,), jnp.float32),
            pltpu.VMEM((128,), jnp.float32),
            pltpu.VMEM((_BPT * 128,), jnp.float32),
            pltpu.VMEM((16,), jnp.float32),
            pltpu.SemaphoreType.DMA,
            pltpu.SemaphoreType.DMA,
        ],
    )(_sc_body)
    return f(h, w_bcast, mt, lin_b)


def kernel(g, h, W, V, lin_w, lin_b):
    del g
    B, N, D = h.shape
    O, H = lin_w.shape
    R = W.shape[1]
    lb2 = lin_b.reshape(1, O)

    mt = pl.pallas_call(
        _mt_block,
        out_shape=jax.ShapeDtypeStruct((R, O), jnp.float32),
    )(V, lin_w)

    w_round = W.astype(jnp.bfloat16).astype(jnp.float32)
    w_bcast = jnp.broadcast_to(w_round[:, :, None], (D, R, 16)).reshape(-1)
    h_flat = h.reshape(B, N * D)
    sc_out = _sc_scores(h_flat, w_bcast, mt.reshape(-1), lin_b).reshape(_B_SC, O)

    n_tc = (B - _B_SC) // _BB
    off = _B_SC // _BB
    tc_out = pl.pallas_call(
        _gin_block,
        grid=(n_tc,),
        in_specs=[
            pl.BlockSpec((_BB, N, D), lambda i: (i + off, 0, 0)),
            pl.BlockSpec((D, R), lambda i: (0, 0)),
            pl.BlockSpec((H, R), lambda i: (0, 0)),
            pl.BlockSpec((O, H), lambda i: (0, 0)),
            pl.BlockSpec((1, O), lambda i: (0, 0)),
        ],
        out_specs=pl.BlockSpec((_BB, O), lambda i: (i, 0)),
        out_shape=jax.ShapeDtypeStruct((B - _B_SC, O), jnp.float32),
    )(h, W, V, lin_w, lb2)

    return jnp.concatenate([sc_out, tc_out], axis=0)


# speedup vs baseline: 1.7393x; 1.7393x over previous
import functools

import jax
import jax.numpy as jnp
from jax import lax
from jax.experimental import pallas as pl
from jax.experimental.pallas import tpu as pltpu
from jax.experimental.pallas import tpu_sc as plsc

_BB = 256
_NTILES = 32
_BPT = 16
_B_SC = _NTILES * _BPT


def _gin_block(h_ref, w_ref, v_ref, lw_ref, lb_ref, out_ref):
    hb = h_ref[:]
    bb, n, d = hb.shape
    feat = jnp.dot(
        hb.reshape(bb * n, d), w_ref[:], preferred_element_type=jnp.float32
    )
    feat = feat.reshape(bb, n, -1)
    k = n
    while k > 1:
        k //= 2
        feat = feat[:, :k, :] * feat[:, k:, :]
    pooled = feat[:, 0, :]
    m = jnp.dot(lw_ref[:], v_ref[:], preferred_element_type=jnp.float32)
    out_ref[:] = (
        jnp.dot(pooled, m.T, preferred_element_type=jnp.float32) + lb_ref[:]
    )


def _mt_block(v_ref, lw_ref, out_ref):
    out_ref[:] = lax.dot_general(
        v_ref[:], lw_ref[:], (((0,), (1,)), ((), ())),
        preferred_element_type=jnp.float32,
    )


def _sc_body(h_hbm, wb_hbm, mt_hbm, lb_hbm, out_hbm,
             hbuf0, hbuf1, wbv, mtv, lbv, sbuf, sc16, sem0, sem1, semw):
    c = lax.axis_index("c")
    s = lax.axis_index("s")
    wid = s * 2 + c
    base = wid * _BPT
    bufs = (hbuf0, hbuf1)
    sems = (sem0, sem1)

    pltpu.async_copy(wb_hbm, wbv, semw).wait()
    pltpu.async_copy(mt_hbm, mtv, semw).wait()
    pltpu.async_copy(lb_hbm, lbv, semw).wait()

    lane = lax.iota(jnp.int32, 16)
    row_g = [lane + 16 * g for g in range(4)]
    zeros16 = jnp.zeros((16,), jnp.int32)
    rot = [(lane + k) & 15 for k in (8, 4, 2, 1)]

    def _round_bf16(x):
        u = plsc.bitcast(x, jnp.uint32)
        u = (u + jnp.uint32(0x8000)) & jnp.uint32(0xFFFF0000)
        return plsc.bitcast(u, jnp.float32)

    def _mac_pass(buf, g0):
        def d_body(d, accs):
            cold = zeros16 + d
            new = list(accs)
            for gi in range(2):
                hv = _round_bf16(plsc.load_gather(buf, [row_g[g0 + gi], cold]))
                for r in range(8):
                    wv = wbv[pl.ds(d * 128 + r * 16, 16)]
                    new[gi * 8 + r] = new[gi * 8 + r] + hv * wv
            return tuple(new)

        return lax.fori_loop(
            0, 128, d_body,
            tuple(jnp.zeros((16,), jnp.float32) for _ in range(16)),
            unroll=4,
        )

    pltpu.async_copy(h_hbm.at[base], bufs[0], sems[0])
    pltpu.async_copy(h_hbm.at[base + 1], bufs[1], sems[1])

    @pl.loop(0, _BPT, step=2)
    def _pair(i):
        for k in range(2):
            buf, sem = bufs[k], sems[k]
            pltpu.make_async_copy(h_hbm.at[base + i + k], buf, sem).wait()
            acc_lo = _mac_pass(buf, 0)
            acc_hi = _mac_pass(buf, 2)
            pvecs = []
            for r in range(8):
                q = acc_lo[r] * acc_lo[8 + r] * acc_hi[r] * acc_hi[8 + r]
                for ridx in rot:
                    sc16[...] = q
                    q = q * plsc.load_gather(sc16, [ridx])
                pvecs.append(q)
            for og in range(8):
                sv = lbv[pl.ds(og * 16, 16)]
                for r in range(8):
                    sv = sv + pvecs[r] * mtv[pl.ds(r * 128 + og * 16, 16)]
                sbuf[pl.ds((i + k) * 128 + og * 16, 16)] = sv
            nxt = i + k + 2

            @pl.when(nxt < _BPT)
            def _():
                pltpu.async_copy(h_hbm.at[base + nxt], buf, sem)

    pltpu.async_copy(sbuf, out_hbm.at[pl.ds(base * 128, _BPT * 128)], semw).wait()


def _sc_scores(h, w_bcast, mt, lin_b):
    f = functools.partial(
        pl.kernel,
        out_type=jax.ShapeDtypeStruct((_B_SC * 128,), jnp.float32),
        mesh=plsc.VectorSubcoreMesh(core_axis_name="c", subcore_axis_name="s"),
        compiler_params=pltpu.CompilerParams(needs_layout_passes=False),
        scratch_types=[
            pltpu.VMEM((64, 128), jnp.float32),
            pltpu.VMEM((64, 128), jnp.float32),
            pltpu.VMEM((128 * 8 * 16,), jnp.float32),
            pltpu.VMEM((8 * 128,), jnp.float32),
            pltpu.VMEM((128,), jnp.float32),
            pltpu.VMEM((_BPT * 128,), jnp.float32),
            pltpu.VMEM((16,), jnp.float32),
            pltpu.SemaphoreType.DMA,
            pltpu.SemaphoreType.DMA,
            pltpu.SemaphoreType.DMA,
        ],
    )(_sc_body)
    return f(h, w_bcast, mt, lin_b)


def kernel(g, h, W, V, lin_w, lin_b):
    del g
    B, N, D = h.shape
    O, H = lin_w.shape
    R = W.shape[1]
    lb2 = lin_b.reshape(1, O)

    mt = pl.pallas_call(
        _mt_block,
        out_shape=jax.ShapeDtypeStruct((R, O), jnp.float32),
    )(V, lin_w)

    w_round = W.astype(jnp.bfloat16).astype(jnp.float32)
    w_bcast = jnp.broadcast_to(w_round[:, :, None], (D, R, 16)).reshape(-1)
    sc_out = _sc_scores(h, w_bcast, mt.reshape(-1), lin_b).reshape(_B_SC, O)

    n_tc = (B - _B_SC) // _BB
    off = _B_SC // _BB
    tc_out = pl.pallas_call(
        _gin_block,
        grid=(n_tc,),
        in_specs=[
            pl.BlockSpec((_BB, N, D), lambda i: (i + off, 0, 0)),
            pl.BlockSpec((D, R), lambda i: (0, 0)),
            pl.BlockSpec((H, R), lambda i: (0, 0)),
            pl.BlockSpec((O, H), lambda i: (0, 0)),
            pl.BlockSpec((1, O), lambda i: (0, 0)),
        ],
        out_specs=pl.BlockSpec((_BB, O), lambda i: (i, 0)),
        out_shape=jax.ShapeDtypeStruct((B - _B_SC, O), jnp.float32),
    )(h, W, V, lin_w, lb2)

    return jnp.concatenate([sc_out, tc_out], axis=0)


# speedup vs baseline: 2.4431x; 1.4046x over previous
import functools

import jax
import jax.numpy as jnp
from jax import lax
from jax.experimental import pallas as pl
from jax.experimental.pallas import tpu as pltpu
from jax.experimental.pallas import tpu_sc as plsc

_BB = 256
_NTILES = 32
_BPT = 8
_B_SC = _NTILES * _BPT


def _gin_block(h_ref, w_ref, v_ref, lw_ref, lb_ref, out_ref):
    hb = h_ref[:]
    bb, n, d = hb.shape
    feat = jnp.dot(
        hb.reshape(bb * n, d), w_ref[:], preferred_element_type=jnp.float32
    )
    feat = feat.reshape(bb, n, -1)
    k = n
    while k > 1:
        k //= 2
        feat = feat[:, :k, :] * feat[:, k:, :]
    pooled = feat[:, 0, :]
    m = jnp.dot(lw_ref[:], v_ref[:], preferred_element_type=jnp.float32)
    out_ref[:] = (
        jnp.dot(pooled, m.T, preferred_element_type=jnp.float32) + lb_ref[:]
    )


def _mt_block(v_ref, lw_ref, out_ref):
    out_ref[:] = lax.dot_general(
        v_ref[:], lw_ref[:], (((0,), (1,)), ((), ())),
        preferred_element_type=jnp.float32,
    )


def _sc_body(h_hbm, wb_hbm, mt_hbm, lb_hbm, out_hbm,
             hbuf0, hbuf1, wbv, mtv, lbv, sbuf, sc16, sem0, sem1, semw):
    c = lax.axis_index("c")
    s = lax.axis_index("s")
    wid = s * 2 + c
    base = wid * _BPT
    bufs = (hbuf0, hbuf1)
    sems = (sem0, sem1)

    pltpu.async_copy(wb_hbm, wbv, semw).wait()
    pltpu.async_copy(mt_hbm, mtv, semw).wait()
    pltpu.async_copy(lb_hbm, lbv, semw).wait()

    lane = lax.iota(jnp.int32, 16)
    row_g = [lane + 16 * g for g in range(4)]
    zeros16 = jnp.zeros((16,), jnp.int32)
    rot = [(lane + k) & 15 for k in (8, 4, 2, 1)]

    def _round_bf16(x):
        u = plsc.bitcast(x, jnp.uint32)
        u = (u + jnp.uint32(0x8000)) & jnp.uint32(0xFFFF0000)
        return plsc.bitcast(u, jnp.float32)

    def _mac_pass(buf, g0):
        def d_body(d, accs):
            cold = zeros16 + d
            new = list(accs)
            for gi in range(2):
                hv = _round_bf16(plsc.load_gather(buf, [row_g[g0 + gi], cold]))
                for r in range(8):
                    wv = wbv[pl.ds(d * 128 + r * 16, 16)]
                    new[gi * 8 + r] = new[gi * 8 + r] + hv * wv
            return tuple(new)

        return lax.fori_loop(
            0, 128, d_body,
            tuple(jnp.zeros((16,), jnp.float32) for _ in range(16)),
            unroll=8,
        )

    pltpu.async_copy(h_hbm.at[base], bufs[0], sems[0])
    pltpu.async_copy(h_hbm.at[base + 1], bufs[1], sems[1])

    @pl.loop(0, _BPT, step=2)
    def _pair(i):
        for k in range(2):
            buf, sem = bufs[k], sems[k]
            pltpu.make_async_copy(h_hbm.at[base + i + k], buf, sem).wait()
            acc_lo = _mac_pass(buf, 0)
            acc_hi = _mac_pass(buf, 2)
            pvecs = []
            for r in range(8):
                q = acc_lo[r] * acc_lo[8 + r] * acc_hi[r] * acc_hi[8 + r]
                for ridx in rot:
                    sc16[...] = q
                    q = q * plsc.load_gather(sc16, [ridx])
                pvecs.append(q)
            for og in range(8):
                sv = lbv[pl.ds(og * 16, 16)]
                for r in range(8):
                    sv = sv + pvecs[r] * mtv[pl.ds(r * 128 + og * 16, 16)]
                sbuf[pl.ds((i + k) * 128 + og * 16, 16)] = sv
            nxt = i + k + 2

            @pl.when(nxt < _BPT)
            def _():
                pltpu.async_copy(h_hbm.at[base + nxt], buf, sem)

    pltpu.async_copy(sbuf, out_hbm.at[pl.ds(base * 128, _BPT * 128)], semw).wait()


def _sc_scores(h, w_bcast, mt, lin_b):
    f = functools.partial(
        pl.kernel,
        out_type=jax.ShapeDtypeStruct((_B_SC * 128,), jnp.float32),
        mesh=plsc.VectorSubcoreMesh(core_axis_name="c", subcore_axis_name="s"),
        compiler_params=pltpu.CompilerParams(needs_layout_passes=False),
        scratch_types=[
            pltpu.VMEM((64, 128), jnp.float32),
            pltpu.VMEM((64, 128), jnp.float32),
            pltpu.VMEM((128 * 8 * 16,), jnp.float32),
            pltpu.VMEM((8 * 128,), jnp.float32),
            pltpu.VMEM((128,), jnp.float32),
            pltpu.VMEM((_BPT * 128,), jnp.float32),
            pltpu.VMEM((16,), jnp.float32),
            pltpu.SemaphoreType.DMA,
            pltpu.SemaphoreType.DMA,
            pltpu.SemaphoreType.DMA,
        ],
    )(_sc_body)
    return f(h, w_bcast, mt, lin_b)


def kernel(g, h, W, V, lin_w, lin_b):
    del g
    B, N, D = h.shape
    O, H = lin_w.shape
    R = W.shape[1]
    lb2 = lin_b.reshape(1, O)

    mt = pl.pallas_call(
        _mt_block,
        out_shape=jax.ShapeDtypeStruct((R, O), jnp.float32),
    )(V, lin_w)

    w_round = W.astype(jnp.bfloat16).astype(jnp.float32)
    w_bcast = jnp.broadcast_to(w_round[:, :, None], (D, R, 16)).reshape(-1)
    sc_out = _sc_scores(h, w_bcast, mt.reshape(-1), lin_b).reshape(_B_SC, O)

    n_tc = (B - _B_SC) // _BB
    off = _B_SC // _BB
    tc_out = pl.pallas_call(
        _gin_block,
        grid=(n_tc,),
        in_specs=[
            pl.BlockSpec((_BB, N, D), lambda i: (i + off, 0, 0)),
            pl.BlockSpec((D, R), lambda i: (0, 0)),
            pl.BlockSpec((H, R), lambda i: (0, 0)),
            pl.BlockSpec((O, H), lambda i: (0, 0)),
            pl.BlockSpec((1, O), lambda i: (0, 0)),
        ],
        out_specs=pl.BlockSpec((_BB, O), lambda i: (i, 0)),
        out_shape=jax.ShapeDtypeStruct((B - _B_SC, O), jnp.float32),
    )(h, W, V, lin_w, lb2)

    return jnp.concatenate([sc_out, tc_out], axis=0)


# speedup vs baseline: 5.2047x; 2.1304x over previous
import jax
import jax.numpy as jnp
from jax.experimental import pallas as pl

_BB = 256
_NCHUNK = 8


def _gin_block(h_ref, w_ref, v_ref, lw_ref, lb_ref, out_ref):
    hb = h_ref[:]
    bb, n, d = hb.shape
    nc = n // _NCHUNK
    w = w_ref[:]
    feats = [
        jnp.dot(
            hb[:, j * nc:(j + 1) * nc, :].reshape(bb * nc, d),
            w,
            preferred_element_type=jnp.float32,
        )
        for j in range(_NCHUNK)
    ]
    m = _NCHUNK
    while m > 1:
        m //= 2
        feats = [feats[j] * feats[j + m] for j in range(m)]
    feat = feats[0].reshape(bb, nc, -1)
    k = nc
    while k > 1:
        k //= 2
        feat = feat[:, :k, :] * feat[:, k:, :]
    pooled = feat[:, 0, :]
    mm = jnp.dot(lw_ref[:], v_ref[:], preferred_element_type=jnp.float32)
    out_ref[:] = (
        jnp.dot(pooled, mm.T, preferred_element_type=jnp.float32) + lb_ref[:]
    )


def kernel(g, h, W, V, lin_w, lin_b):
    del g
    B, N, D = h.shape
    O, H = lin_w.shape
    R = W.shape[1]
    lb2 = lin_b.reshape(1, O)
    grid = (B // _BB,)
    return pl.pallas_call(
        _gin_block,
        grid=grid,
        in_specs=[
            pl.BlockSpec((_BB, N, D), lambda i: (i, 0, 0)),
            pl.BlockSpec((D, R), lambda i: (0, 0)),
            pl.BlockSpec((H, R), lambda i: (0, 0)),
            pl.BlockSpec((O, H), lambda i: (0, 0)),
            pl.BlockSpec((1, O), lambda i: (0, 0)),
        ],
        out_specs=pl.BlockSpec((_BB, O), lambda i: (i, 0)),
        out_shape=jax.ShapeDtypeStruct((B, O), jnp.float32),
    )(h, W, V, lin_w, lb2)
